# baseline (speedup 1.0000x reference)
import functools

import jax
import jax.numpy as jnp
from jax import lax
from jax.experimental import pallas as pl
from jax.experimental.pallas import tpu as pltpu
from jax.experimental.pallas import tpu_sc as plsc

N_POIS = 10000
EMB = 128
NNZ = 320000

NC = 2
NS = 16
NW = NC * NS
EPW = NNZ // NW
K = 112
HK = K // 2
NCHUNKS = -(-EPW // K)
NTRIS = 30
NCH_TOT = 2 + 3 * NTRIS
EPAD = NNZ + 1024
ROWS_OUT = 10240
SROWS = 10256
RPW_C = ROWS_OUT // NW
RRPW = ROWS_OUT // 16
PADROW = ROWS_OUT
LANES = EMB // 16


def _scale16(gath_b, vbuf_b, base_rel, j0, e1_rel, iota16):
    vv = vbuf_b[pl.ds(j0, 16)]
    eidx = base_rel + j0 + iota16
    vvz = jnp.where(eidx < e1_rel, vv, 0.0)
    for lane in range(16):
        vb = vvz[iota16 * 0 + lane]
        jrow = j0 + lane
        for cc in range(LANES):
            sl = pl.ds(cc * 16, 16)
            gath_b[jrow, sl] = gath_b[jrow, sl] * vb
    return None


def _spmm_partial_body(dense, rows, cols, vals, out2,
                       cbuf0, cbuf1, cbuf2, rbuf0, rbuf1, rbuf2,
                       vbuf0, vbuf1, vbuf2, gath0, gath1, gath2, sacc,
                       semc0, semc1, semc2, semr0, semr1, semr2,
                       semv0, semv1, semv2, semg0, semg1, semg2,
                       sems0, sems1, sems2):
    c = lax.axis_index("c")
    s = lax.axis_index("s")
    wid = c * NS + s
    e0 = wid * EPW

    cbuf = (cbuf0, cbuf1, cbuf2)
    rbuf = (rbuf0, rbuf1, rbuf2)
    vbuf = (vbuf0, vbuf1, vbuf2)
    gath = (gath0, gath1, gath2)
    semc = (semc0, semc1, semc2)
    semr = (semr0, semr1, semr2)
    semv = (semv0, semv1, semv2)
    semg = (semg0, semg1, semg2)
    sems = (sems0, sems1, sems2)
    iota16 = lax.iota(jnp.int32, 16)

    zero = jnp.zeros((16,), jnp.float32)

    def zb(r, carry):
        for cc in range(LANES):
            gath0[r, pl.ds(cc * 16, 16)] = zero
        return carry

    lax.fori_loop(0, K, zb, 0)
    for z in range(5):
        pltpu.sync_copy(gath0, sacc.at[pl.ds(s * RRPW + z * K, K)])
    pltpu.sync_copy(gath0.at[pl.ds(0, RRPW - 5 * K)],
                    sacc.at[pl.ds(s * RRPW + 5 * K, RRPW - 5 * K)])
    plsc.subcore_barrier()

    def cbase(ci):
        return pl.multiple_of(e0 + ci * K, 8)

    def start_cols(ci, b):
        pltpu.make_async_copy(cols.at[pl.ds(cbase(ci), K)], cbuf[b], semc[b]).start()

    def start_rv(ci, b):
        for h in (0, 1):
            pltpu.make_async_copy(rows.at[pl.ds(cbase(ci) + h * HK, HK)],
                                  rbuf[b].at[h], semr[b]).start()
        pltpu.make_async_copy(vals.at[pl.ds(cbase(ci), K)], vbuf[b], semv[b]).start()

    def start_gather(b):
        for h in (0, 1):
            pltpu.make_async_copy(dense.at[cbuf[b].at[pl.ds(h * HK, HK)]],
                                  gath[b].at[pl.ds(h * HK, HK)], semg[b]).start()

    def start_scatter(b):
        for h in (0, 1):
            pltpu.make_async_copy(gath[b].at[pl.ds(h * HK, HK)],
                                  sacc.at[rbuf[b].at[h]], sems[b]).start(add=True)

    def wait_cols(b):
        pltpu.make_async_copy(cols.at[pl.ds(0, K)], cbuf[b], semc[b]).wait()

    def wait_rv(b):
        for h in (0, 1):
            pltpu.make_async_copy(rows.at[pl.ds(0, HK)], rbuf[b].at[h],
                                  semr[b]).wait()
        pltpu.make_async_copy(vals.at[pl.ds(0, K)], vbuf[b], semv[b]).wait()

    def wait_gather(b):
        for h in (0, 1):
            pltpu.make_async_copy(dense.at[cbuf[b].at[pl.ds(0, HK)]],
                                  gath[b].at[pl.ds(h * HK, HK)], semg[b]).wait()

    def wait_scatter(b):
        for h in (0, 1):
            pltpu.make_async_copy(gath[b].at[pl.ds(h * HK, HK)],
                                  sacc.at[rbuf[b].at[h]], sems[b]).wait()

    def scale(ci, b):
        base_rel = ci * K

        def group_body(g16, gcarry):
            _scale16(gath[b], vbuf[b], base_rel, g16 * 16, EPW, iota16)
            return gcarry

        lax.fori_loop(0, K // 16, group_body, 0)

    def iter_steps(ci, b, first):
        if not first:
            wait_scatter((b + 1) % 3)
        wait_cols((b + 1) % 3)
        start_gather((b + 1) % 3)
        start_rv(ci + 1, (b + 1) % 3)
        wait_gather(b)
        wait_rv(b)
        scale(ci, b)
        start_scatter(b)
        start_cols(ci + 2, (b + 2) % 3)

    start_cols(0, 0)
    wait_cols(0)
    start_gather(0)
    start_rv(0, 0)
    start_cols(1, 1)

    iter_steps(0, 0, True)
    iter_steps(1, 1, True)

    def tri_body(p, carry):
        ci = 2 + p * 3
        iter_steps(ci, 2, False)
        iter_steps(ci + 1, 0, False)
        iter_steps(ci + 2, 1, False)
        return carry

    lax.fori_loop(0, NTRIS, tri_body, 0)

    wait_scatter(0)
    wait_scatter(1)
    wait_gather(2)
    wait_rv(2)
    wait_cols(0)

    plsc.subcore_barrier()
    pltpu.sync_copy(sacc.at[pl.ds(s * RRPW, RRPW)],
                    out2.at[c, pl.ds(s * RRPW, RRPW)])


def _combine_body(in2, out, buf0, buf1, sem0, sem1):
    c = lax.axis_index("c")
    s = lax.axis_index("s")
    wid = c * NS + s
    r0 = wid * RPW_C
    cp0 = pltpu.make_async_copy(in2.at[0, pl.ds(r0, RPW_C)], buf0, sem0)
    cp1 = pltpu.make_async_copy(in2.at[1, pl.ds(r0, RPW_C)], buf1, sem1)
    cp0.start()
    cp1.start()
    cp0.wait()
    cp1.wait()

    def add_row(r, carry):
        for cc in range(LANES):
            sl = pl.ds(cc * 16, 16)
            buf0[r, sl] = buf0[r, sl] + buf1[r, sl]
        return carry

    lax.fori_loop(0, RPW_C, add_row, 0)
    pltpu.sync_copy(buf0, out.at[pl.ds(r0, RPW_C)])


@functools.cache
def _spmm_partial_kernel(n_dense_rows):
    mesh = plsc.VectorSubcoreMesh(core_axis_name="c", subcore_axis_name="s")
    return pl.kernel(
        _spmm_partial_body,
        mesh=mesh,
        out_type=jax.ShapeDtypeStruct((NC, ROWS_OUT, EMB), jnp.float32),
        scratch_types=[
            pltpu.VMEM((K,), jnp.int32),
            pltpu.VMEM((K,), jnp.int32),
            pltpu.VMEM((K,), jnp.int32),
            pltpu.VMEM((2, HK), jnp.int32),
            pltpu.VMEM((2, HK), jnp.int32),
            pltpu.VMEM((2, HK), jnp.int32),
            pltpu.VMEM((K,), jnp.float32),
            pltpu.VMEM((K,), jnp.float32),
            pltpu.VMEM((K,), jnp.float32),
            pltpu.VMEM((K, EMB), jnp.float32),
            pltpu.VMEM((K, EMB), jnp.float32),
            pltpu.VMEM((K, EMB), jnp.float32),
            pltpu.VMEM_SHARED((SROWS, EMB), jnp.float32),
        ] + [pltpu.SemaphoreType.DMA] * 15,
    )


@functools.cache
def _combine_kernel():
    mesh = plsc.VectorSubcoreMesh(core_axis_name="c", subcore_axis_name="s")
    return pl.kernel(
        _combine_body,
        mesh=mesh,
        out_type=jax.ShapeDtypeStruct((ROWS_OUT, EMB), jnp.float32),
        scratch_types=[
            pltpu.VMEM((RPW_C, EMB), jnp.float32),
            pltpu.VMEM((RPW_C, EMB), jnp.float32),
            pltpu.SemaphoreType.DMA,
            pltpu.SemaphoreType.DMA,
        ],
    )


def _spmm(dense, rows, cols, vals):
    pad = EPAD - NNZ
    rows_p = jnp.concatenate(
        [rows.astype(jnp.int32), jnp.full((pad,), PADROW, jnp.int32)])
    cols_p = jnp.concatenate([cols.astype(jnp.int32), jnp.zeros((pad,), jnp.int32)])
    vals_p = jnp.concatenate([vals, jnp.zeros((pad,), jnp.float32)])
    partials = _spmm_partial_kernel(dense.shape[0])(dense, rows_p, cols_p, vals_p)
    return _combine_kernel()(partials)


def kernel(pois_embs, hg_up_rows, hg_up_cols, hg_up_vals,
           hg_pu_rows, hg_pu_cols, hg_pu_vals):
    msg = _spmm(pois_embs, hg_up_rows, hg_up_cols, hg_up_vals)
    out = _spmm(msg, hg_pu_rows, hg_pu_cols, hg_pu_vals)
    return out[:N_POIS]


# speedup vs baseline: 1.0024x; 1.0024x over previous
import functools

import jax
import jax.numpy as jnp
from jax import lax
from jax.experimental import pallas as pl
from jax.experimental.pallas import tpu as pltpu
from jax.experimental.pallas import tpu_sc as plsc

N_POIS = 10000
EMB = 128
NNZ = 320000

NC = 2
NS = 16
NW = NC * NS
EPW = NNZ // NW
K = 112
NCHUNKS = -(-EPW // K)
NTRIS = 30
NCH_TOT = 2 + 3 * NTRIS
EPAD = NNZ + 1024
ROWS_OUT = 10240
SROWS = 10256
RPW_C = ROWS_OUT // NW
RRPW = ROWS_OUT // 16
PADROW = ROWS_OUT
LANES = EMB // 16


def _scale16(gath_b, vbuf_b, base_rel, j0, e1_rel, iota16):
    vv = vbuf_b[pl.ds(j0, 16)]
    eidx = base_rel + j0 + iota16
    vvz = jnp.where(eidx < e1_rel, vv, 0.0)
    for lane in range(16):
        vb = vvz[iota16 * 0 + lane]
        jrow = j0 + lane
        for cc in range(LANES):
            sl = pl.ds(cc * 16, 16)
            gath_b[jrow, sl] = gath_b[jrow, sl] * vb
    return None


def _spmm_partial_body(dense, rows, cols, vals, out2,
                       cbuf0, cbuf1, cbuf2, rbuf0, rbuf1, rbuf2,
                       vbuf0, vbuf1, vbuf2, gath0, gath1, gath2, sacc,
                       semc0, semc1, semc2, semr0, semr1, semr2,
                       semv0, semv1, semv2, semg0, semg1, semg2,
                       sems0, sems1, sems2):
    c = lax.axis_index("c")
    s = lax.axis_index("s")
    wid = c * NS + s
    e0 = wid * EPW

    cbuf = (cbuf0, cbuf1, cbuf2)
    rbuf = (rbuf0, rbuf1, rbuf2)
    vbuf = (vbuf0, vbuf1, vbuf2)
    gath = (gath0, gath1, gath2)
    semc = (semc0, semc1, semc2)
    semr = (semr0, semr1, semr2)
    semv = (semv0, semv1, semv2)
    semg = (semg0, semg1, semg2)
    sems = (sems0, sems1, sems2)
    iota16 = lax.iota(jnp.int32, 16)

    zero = jnp.zeros((16,), jnp.float32)

    def zb(r, carry):
        for cc in range(LANES):
            gath0[r, pl.ds(cc * 16, 16)] = zero
        return carry

    lax.fori_loop(0, K, zb, 0)
    for z in range(5):
        pltpu.sync_copy(gath0, sacc.at[pl.ds(s * RRPW + z * K, K)])
    pltpu.sync_copy(gath0.at[pl.ds(0, RRPW - 5 * K)],
                    sacc.at[pl.ds(s * RRPW + 5 * K, RRPW - 5 * K)])
    plsc.subcore_barrier()

    def cbase(ci):
        return pl.multiple_of(e0 + ci * K, 8)

    def start_cols(ci, b):
        pltpu.make_async_copy(cols.at[pl.ds(cbase(ci), K)], cbuf[b], semc[b]).start()

    def start_rv(ci, b):
        pltpu.make_async_copy(rows.at[pl.ds(cbase(ci), K)], rbuf[b], semr[b]).start()
        pltpu.make_async_copy(vals.at[pl.ds(cbase(ci), K)], vbuf[b], semv[b]).start()

    def start_gather(b):
        pltpu.make_async_copy(dense.at[cbuf[b]], gath[b], semg[b]).start()

    def start_scatter(b):
        pltpu.make_async_copy(gath[b], sacc.at[rbuf[b]], sems[b]).start(add=True)

    def wait_cols(b):
        pltpu.make_async_copy(cols.at[pl.ds(0, K)], cbuf[b], semc[b]).wait()

    def wait_rv(b):
        pltpu.make_async_copy(rows.at[pl.ds(0, K)], rbuf[b], semr[b]).wait()
        pltpu.make_async_copy(vals.at[pl.ds(0, K)], vbuf[b], semv[b]).wait()

    def wait_gather(b):
        pltpu.make_async_copy(dense.at[cbuf[b]], gath[b], semg[b]).wait()

    def wait_scatter(b):
        pltpu.make_async_copy(gath[b], sacc.at[rbuf[b]], sems[b]).wait()

    def scale(ci, b):
        base_rel = ci * K

        def group_body(g16, gcarry):
            _scale16(gath[b], vbuf[b], base_rel, g16 * 16, EPW, iota16)
            return gcarry

        lax.fori_loop(0, K // 16, group_body, 0)

    def iter_steps(ci, b, first):
        if not first:
            wait_scatter((b + 1) % 3)
        wait_cols((b + 1) % 3)
        start_gather((b + 1) % 3)
        start_rv(ci + 1, (b + 1) % 3)
        wait_gather(b)
        wait_rv(b)
        scale(ci, b)
        start_scatter(b)
        start_cols(ci + 2, (b + 2) % 3)

    start_cols(0, 0)
    wait_cols(0)
    start_gather(0)
    start_rv(0, 0)
    start_cols(1, 1)

    iter_steps(0, 0, True)
    iter_steps(1, 1, True)

    def tri_body(p, carry):
        ci = 2 + p * 3
        iter_steps(ci, 2, False)
        iter_steps(ci + 1, 0, False)
        iter_steps(ci + 2, 1, False)
        return carry

    lax.fori_loop(0, NTRIS, tri_body, 0)

    wait_scatter(0)
    wait_scatter(1)
    wait_gather(2)
    wait_rv(2)
    wait_cols(0)

    plsc.subcore_barrier()
    pltpu.sync_copy(sacc.at[pl.ds(s * RRPW, RRPW)],
                    out2.at[c, pl.ds(s * RRPW, RRPW)])


def _combine_body(in2, out, buf0, buf1, sem0, sem1):
    c = lax.axis_index("c")
    s = lax.axis_index("s")
    wid = c * NS + s
    r0 = wid * RPW_C
    cp0 = pltpu.make_async_copy(in2.at[0, pl.ds(r0, RPW_C)], buf0, sem0)
    cp1 = pltpu.make_async_copy(in2.at[1, pl.ds(r0, RPW_C)], buf1, sem1)
    cp0.start()
    cp1.start()
    cp0.wait()
    cp1.wait()

    def add_row(r, carry):
        for cc in range(LANES):
            sl = pl.ds(cc * 16, 16)
            buf0[r, sl] = buf0[r, sl] + buf1[r, sl]
        return carry

    lax.fori_loop(0, RPW_C, add_row, 0)
    pltpu.sync_copy(buf0, out.at[pl.ds(r0, RPW_C)])


@functools.cache
def _spmm_partial_kernel(n_dense_rows):
    mesh = plsc.VectorSubcoreMesh(core_axis_name="c", subcore_axis_name="s")
    return pl.kernel(
        _spmm_partial_body,
        mesh=mesh,
        out_type=jax.ShapeDtypeStruct((NC, ROWS_OUT, EMB), jnp.float32),
        scratch_types=[
            pltpu.VMEM((K,), jnp.int32),
            pltpu.VMEM((K,), jnp.int32),
            pltpu.VMEM((K,), jnp.int32),
            pltpu.VMEM((K,), jnp.int32),
            pltpu.VMEM((K,), jnp.int32),
            pltpu.VMEM((K,), jnp.int32),
            pltpu.VMEM((K,), jnp.float32),
            pltpu.VMEM((K,), jnp.float32),
            pltpu.VMEM((K,), jnp.float32),
            pltpu.VMEM((K, EMB), jnp.float32),
            pltpu.VMEM((K, EMB), jnp.float32),
            pltpu.VMEM((K, EMB), jnp.float32),
            pltpu.VMEM_SHARED((SROWS, EMB), jnp.float32),
        ] + [pltpu.SemaphoreType.DMA] * 15,
    )


@functools.cache
def _combine_kernel():
    mesh = plsc.VectorSubcoreMesh(core_axis_name="c", subcore_axis_name="s")
    return pl.kernel(
        _combine_body,
        mesh=mesh,
        out_type=jax.ShapeDtypeStruct((ROWS_OUT, EMB), jnp.float32),
        scratch_types=[
            pltpu.VMEM((RPW_C, EMB), jnp.float32),
            pltpu.VMEM((RPW_C, EMB), jnp.float32),
            pltpu.SemaphoreType.DMA,
            pltpu.SemaphoreType.DMA,
        ],
    )


def _spmm(dense, rows, cols, vals):
    pad = EPAD - NNZ
    rows_p = jnp.concatenate(
        [rows.astype(jnp.int32), jnp.full((pad,), PADROW, jnp.int32)])
    cols_p = jnp.concatenate([cols.astype(jnp.int32), jnp.zeros((pad,), jnp.int32)])
    vals_p = jnp.concatenate([vals, jnp.zeros((pad,), jnp.float32)])
    partials = _spmm_partial_kernel(dense.shape[0])(dense, rows_p, cols_p, vals_p)
    return _combine_kernel()(partials)


def kernel(pois_embs, hg_up_rows, hg_up_cols, hg_up_vals,
           hg_pu_rows, hg_pu_cols, hg_pu_vals):
    msg = _spmm(pois_embs, hg_up_rows, hg_up_cols, hg_up_vals)
    out = _spmm(msg, hg_pu_rows, hg_pu_cols, hg_pu_vals)
    return out[:N_POIS]
